# Initial kernel scaffold; baseline (speedup 1.0000x reference)
import jax
import jax.numpy as jnp
from jax.experimental import pallas as pl


def kernel(x, edge_index, W1, b1, W2, b2):
    raise NotImplementedError("write your pallas kernel here")


# speedup vs baseline: 19.1912x; 19.1912x over previous
import functools

import jax
import jax.numpy as jnp
from jax import lax
from jax.experimental import pallas as pl
from jax.experimental.pallas import tpu as pltpu
from jax.experimental.pallas import tpu_sc as plsc

N = 10000
NP = 10240
E = 320000
F_IN = 128
HID = 128
NCLS = 40
CPAD = 128

NC = 2
NS = 16
NW = NC * NS
EPW = E // NW
K = 80
NB = EPW // K
RPT = NP // NS
ZB = 128
ZBA = 32

_MESH = dict(core_axis_name="c", subcore_axis_name="s")


def _sc_degree(dst_r):
    @functools.partial(
        pl.kernel,
        out_type=jax.ShapeDtypeStruct((NC, NP, 16), jnp.float32),
        mesh=plsc.VectorSubcoreMesh(**_MESH),
        scratch_types=[
            pltpu.VMEM((NB, K), jnp.int32),
            pltpu.VMEM((K, 16), jnp.float32),
            pltpu.VMEM((ZB, 16), jnp.float32),
            pltpu.VMEM_SHARED((NP, 16), jnp.float32),
        ],
    )
    def k(dst_hbm, out_hbm, dst_i, ones_v, zbuf, acc):
        c = lax.axis_index("c")
        s = lax.axis_index("s")
        wid = s * NC + c
        one = jnp.ones((16,), jnp.float32)
        zero = jnp.zeros((16,), jnp.float32)

        def fill_ones(i, _):
            ones_v[i, :] = one
            return 0

        lax.fori_loop(0, K, fill_ones, 0)

        def fill_zero(i, _):
            zbuf[i, :] = zero
            return 0

        lax.fori_loop(0, ZB, fill_zero, 0)
        for t in range(RPT // ZB):
            pltpu.sync_copy(zbuf, acc.at[pl.ds(s * RPT + t * ZB, ZB)])
        plsc.subcore_barrier()

        pltpu.sync_copy(dst_hbm.at[wid], dst_i)

        def body(b, _):
            pltpu.sync_copy(ones_v, acc.at[dst_i.at[b]], add=True)
            return 0

        lax.fori_loop(0, NB, body, 0)
        plsc.subcore_barrier()
        pltpu.sync_copy(
            acc.at[pl.ds(s * RPT, RPT)], out_hbm.at[c, pl.ds(s * RPT, RPT)]
        )

    return k(dst_r)


def _sc_aggregate(y, src_r, dst_r, d):
    @functools.partial(
        pl.kernel,
        out_type=jax.ShapeDtypeStruct((NC, NP, d), jnp.float32),
        mesh=plsc.VectorSubcoreMesh(**_MESH),
        scratch_types=[
            pltpu.VMEM((NB, K), jnp.int32),
            pltpu.VMEM((NB, K), jnp.int32),
            pltpu.VMEM((K, d), jnp.float32),
            pltpu.VMEM((ZBA, d), jnp.float32),
            pltpu.VMEM_SHARED((NP, d), jnp.float32),
            pltpu.SemaphoreType.DMA,
        ],
    )
    def k(y_hbm, src_hbm, dst_hbm, out_hbm, src_i, dst_i, rows, zbuf, acc, sem):
        c = lax.axis_index("c")
        s = lax.axis_index("s")
        wid = s * NC + c
        zero = jnp.zeros((16,), jnp.float32)

        def fill_zero(i, _):
            for g in range(d // 16):
                zbuf[i, pl.ds(g * 16, 16)] = zero
            return 0

        lax.fori_loop(0, ZBA, fill_zero, 0)

        def zero_acc(t, _):
            pltpu.sync_copy(zbuf, acc.at[pl.ds(s * RPT + t * ZBA, ZBA)])
            return 0

        lax.fori_loop(0, RPT // ZBA, zero_acc, 0)
        plsc.subcore_barrier()

        pltpu.sync_copy(src_hbm.at[wid], src_i)
        pltpu.sync_copy(dst_hbm.at[wid], dst_i)

        def body(b, _):
            pltpu.async_copy(y_hbm.at[src_i.at[b]], rows, sem).wait()
            pltpu.sync_copy(rows, acc.at[dst_i.at[b]], add=True)
            return 0

        lax.fori_loop(0, NB, body, 0)
        plsc.subcore_barrier()
        pltpu.sync_copy(
            acc.at[pl.ds(s * RPT, RPT)], out_hbm.at[c, pl.ds(s * RPT, RPT)]
        )

    return k(y, src_r, dst_r)


BLK = 1024


def _dis_from(degp_ref):
    dp = degp_ref[0] + degp_ref[1]
    return lax.rsqrt(dp[:, 0:1] + 1.0)


def _tc_layer1(x, W1, degp):
    def body(x_ref, w_ref, degp_ref, y_ref):
        dis = _dis_from(degp_ref)
        y_ref[...] = (
            jnp.dot(x_ref[...], w_ref[...], preferred_element_type=jnp.float32)
            * dis
        )

    return pl.pallas_call(
        body,
        grid=(NP // BLK,),
        in_specs=[
            pl.BlockSpec((BLK, F_IN), lambda i: (i, 0)),
            pl.BlockSpec((F_IN, HID), lambda i: (0, 0)),
            pl.BlockSpec((NC, BLK, 16), lambda i: (0, i, 0)),
        ],
        out_specs=pl.BlockSpec((BLK, HID), lambda i: (i, 0)),
        out_shape=jax.ShapeDtypeStruct((NP, HID), jnp.float32),
    )(x, W1, degp)


def _tc_layer2_in(acc1, y1, degp, b1, W2p):
    def body(acc_ref, y1_ref, degp_ref, b1_ref, w2_ref, y2_ref):
        dis = _dis_from(degp_ref)
        ssum = acc_ref[0] + acc_ref[1] + y1_ref[...]
        h = jnp.maximum(ssum * dis + b1_ref[...], 0.0)
        y2_ref[...] = (
            jnp.dot(h, w2_ref[...], preferred_element_type=jnp.float32) * dis
        )

    return pl.pallas_call(
        body,
        grid=(NP // BLK,),
        in_specs=[
            pl.BlockSpec((NC, BLK, HID), lambda i: (0, i, 0)),
            pl.BlockSpec((BLK, HID), lambda i: (i, 0)),
            pl.BlockSpec((NC, BLK, 16), lambda i: (0, i, 0)),
            pl.BlockSpec((1, HID), lambda i: (0, 0)),
            pl.BlockSpec((HID, CPAD), lambda i: (0, 0)),
        ],
        out_specs=pl.BlockSpec((BLK, CPAD), lambda i: (i, 0)),
        out_shape=jax.ShapeDtypeStruct((NP, CPAD), jnp.float32),
    )(acc1, y1, degp, b1, W2p)


def _tc_final(acc2, y2, degp, b2p):
    def body(acc_ref, y2_ref, degp_ref, b2_ref, out_ref):
        dis = _dis_from(degp_ref)
        out_ref[...] = (
            acc_ref[0] + acc_ref[1] + y2_ref[...]
        ) * dis + b2_ref[...]

    return pl.pallas_call(
        body,
        grid=(NP // BLK,),
        in_specs=[
            pl.BlockSpec((NC, BLK, CPAD), lambda i: (0, i, 0)),
            pl.BlockSpec((BLK, CPAD), lambda i: (i, 0)),
            pl.BlockSpec((NC, BLK, 16), lambda i: (0, i, 0)),
            pl.BlockSpec((1, CPAD), lambda i: (0, 0)),
        ],
        out_specs=pl.BlockSpec((BLK, CPAD), lambda i: (i, 0)),
        out_shape=jax.ShapeDtypeStruct((NP, CPAD), jnp.float32),
    )(acc2, y2, degp, b2p)


def kernel(x, edge_index, W1, b1, W2, b2):
    src_r = edge_index[0].reshape(NW, NB, K)
    dst_r = edge_index[1].reshape(NW, NB, K)
    W2p = jnp.pad(W2, ((0, 0), (0, CPAD - NCLS)))
    b1r = b1.reshape(1, HID)
    b2p = jnp.pad(b2, (0, CPAD - NCLS)).reshape(1, CPAD)

    xp = jnp.pad(x, ((0, NP - N), (0, 0)))
    degp = _sc_degree(dst_r)
    y1 = _tc_layer1(xp, W1, degp)
    acc1 = _sc_aggregate(y1, src_r, dst_r, HID)
    y2 = _tc_layer2_in(acc1, y1, degp, b1r, W2p)
    acc2 = _sc_aggregate(y2, src_r, dst_r, CPAD)
    outp = _tc_final(acc2, y2, degp, b2p)
    return outp[:N, :NCLS]


# speedup vs baseline: 32.0349x; 1.6692x over previous
import functools

import jax
import jax.numpy as jnp
from jax import lax
from jax.experimental import pallas as pl
from jax.experimental.pallas import tpu as pltpu
from jax.experimental.pallas import tpu_sc as plsc

N = 10000
NP = 10240
E = 320000
F_IN = 128
HID = 128
NCLS = 40
CPAD = 48

NC = 2
NS = 16
NW = NC * NS
EPW = E // NW
K = 80
NB = EPW // K
RPT = NP // NS
ZB = 128
ZBA = 32

_MESH = dict(core_axis_name="c", subcore_axis_name="s")


def _sc_degree(dst_r):
    @functools.partial(
        pl.kernel,
        out_type=jax.ShapeDtypeStruct((NC, NP, 16), jnp.float32),
        mesh=plsc.VectorSubcoreMesh(**_MESH),
        compiler_params=pltpu.CompilerParams(use_tc_tiling_on_sc=False),
        scratch_types=[
            pltpu.VMEM((NB, K), jnp.int32),
            pltpu.VMEM((K, 16), jnp.float32),
            pltpu.VMEM((ZB, 16), jnp.float32),
            pltpu.VMEM_SHARED((NP, 16), jnp.float32),
        ],
    )
    def k(dst_hbm, out_hbm, dst_i, ones_v, zbuf, acc):
        c = lax.axis_index("c")
        s = lax.axis_index("s")
        wid = s * NC + c
        one = jnp.ones((16,), jnp.float32)
        zero = jnp.zeros((16,), jnp.float32)

        def fill_ones(i, _):
            ones_v[i, :] = one
            return 0

        lax.fori_loop(0, K, fill_ones, 0)

        def fill_zero(i, _):
            zbuf[i, :] = zero
            return 0

        lax.fori_loop(0, ZB, fill_zero, 0)
        for t in range(RPT // ZB):
            pltpu.sync_copy(zbuf, acc.at[pl.ds(s * RPT + t * ZB, ZB)])
        plsc.subcore_barrier()

        pltpu.sync_copy(dst_hbm.at[wid], dst_i)

        def body(b, _):
            pltpu.sync_copy(ones_v, acc.at[dst_i.at[b]], add=True)
            return 0

        lax.fori_loop(0, NB, body, 0)
        plsc.subcore_barrier()
        pltpu.sync_copy(
            acc.at[pl.ds(s * RPT, RPT)], out_hbm.at[c, pl.ds(s * RPT, RPT)]
        )

    return k(dst_r)


def _sc_aggregate(y, src_r, dst_r, d):
    @functools.partial(
        pl.kernel,
        out_type=jax.ShapeDtypeStruct((NC, NP, d), jnp.float32),
        mesh=plsc.VectorSubcoreMesh(**_MESH),
        compiler_params=pltpu.CompilerParams(use_tc_tiling_on_sc=False),
        scratch_types=[
            pltpu.VMEM((1, EPW), jnp.int32),
            pltpu.VMEM((NB, K), jnp.int32),
            pltpu.VMEM((K, d), jnp.float32),
            pltpu.VMEM((K, d), jnp.float32),
            pltpu.VMEM_SHARED((NP, d), jnp.float32),
            pltpu.SemaphoreType.DMA,
            pltpu.SemaphoreType.DMA,
        ],
    )
    def k(y_hbm, src_hbm, dst_hbm, out_hbm, src_i, dst_i, rows0, rows1,
          acc, sem0, sem1):
        c = lax.axis_index("c")
        s = lax.axis_index("s")
        wid = s * NC + c
        zero = jnp.zeros((16,), jnp.float32)

        def fill_zero(i, _):
            for g in range(d // 16):
                rows0[i, pl.ds(g * 16, 16)] = zero
            return 0

        lax.fori_loop(0, K, fill_zero, 0)

        def zero_acc(t, _):
            pltpu.sync_copy(rows0, acc.at[pl.ds(s * RPT + t * K, K)])
            return 0

        lax.fori_loop(0, RPT // K, zero_acc, 0)
        plsc.subcore_barrier()

        pltpu.sync_copy(src_hbm.at[wid], src_i)
        pltpu.sync_copy(dst_hbm.at[wid], dst_i)

        bufs = (rows0, rows1)
        sems = (sem0, sem1)

        def gather(b, j):
            pltpu.async_copy(y_hbm.at[src_i.at[0, pl.ds(b * K, K)]], bufs[j], sems[j])

        def wait_scatter(b, j):
            pltpu.make_async_copy(
                y_hbm.at[src_i.at[0, pl.ds(b * K, K)]], bufs[j], sems[j]
            ).wait()
            pltpu.sync_copy(bufs[j], acc.at[dst_i.at[b]], add=True)

        gather(0, 0)

        def body(g2, _):
            b = g2 * 2
            gather(b + 1, 1)
            wait_scatter(b, 0)
            gather(b + 2, 0)
            wait_scatter(b + 1, 1)
            return 0

        lax.fori_loop(0, (NB - 1) // 2, body, 0)
        wait_scatter(NB - 1, 0)
        plsc.subcore_barrier()
        pltpu.sync_copy(
            acc.at[pl.ds(s * RPT, RPT)], out_hbm.at[c, pl.ds(s * RPT, RPT)]
        )

    return k(y, src_r, dst_r)


BLK = 1024


def _dis_from(degp_ref):
    dp = degp_ref[0] + degp_ref[1]
    return lax.rsqrt(dp[:, 0:1] + 1.0)


def _tc_layer1(x, W1, degp):
    def body(x_ref, w_ref, degp_ref, y_ref):
        dis = _dis_from(degp_ref)
        y_ref[...] = (
            jnp.dot(x_ref[...], w_ref[...], preferred_element_type=jnp.float32)
            * dis
        )

    return pl.pallas_call(
        body,
        grid=(NP // BLK,),
        in_specs=[
            pl.BlockSpec((BLK, F_IN), lambda i: (i, 0)),
            pl.BlockSpec((F_IN, HID), lambda i: (0, 0)),
            pl.BlockSpec((NC, BLK, 16), lambda i: (0, i, 0)),
        ],
        out_specs=pl.BlockSpec((BLK, HID), lambda i: (i, 0)),
        out_shape=jax.ShapeDtypeStruct((NP, HID), jnp.float32),
    )(x, W1, degp)


def _tc_layer2_in(acc1, y1, degp, b1, W2p):
    def body(acc_ref, y1_ref, degp_ref, b1_ref, w2_ref, y2_ref):
        dis = _dis_from(degp_ref)
        ssum = acc_ref[0] + acc_ref[1] + y1_ref[...]
        h = jnp.maximum(ssum * dis + b1_ref[...], 0.0)
        y2_ref[...] = (
            jnp.dot(h, w2_ref[...], preferred_element_type=jnp.float32) * dis
        )

    return pl.pallas_call(
        body,
        grid=(NP // BLK,),
        in_specs=[
            pl.BlockSpec((NC, BLK, HID), lambda i: (0, i, 0)),
            pl.BlockSpec((BLK, HID), lambda i: (i, 0)),
            pl.BlockSpec((NC, BLK, 16), lambda i: (0, i, 0)),
            pl.BlockSpec((1, HID), lambda i: (0, 0)),
            pl.BlockSpec((HID, CPAD), lambda i: (0, 0)),
        ],
        out_specs=pl.BlockSpec((BLK, CPAD), lambda i: (i, 0)),
        out_shape=jax.ShapeDtypeStruct((NP, CPAD), jnp.float32),
    )(acc1, y1, degp, b1, W2p)


def _tc_final(acc2, y2, degp, b2p):
    def body(acc_ref, y2_ref, degp_ref, b2_ref, out_ref):
        dis = _dis_from(degp_ref)
        out_ref[...] = (
            acc_ref[0] + acc_ref[1] + y2_ref[...]
        ) * dis + b2_ref[...]

    return pl.pallas_call(
        body,
        grid=(NP // BLK,),
        in_specs=[
            pl.BlockSpec((NC, BLK, CPAD), lambda i: (0, i, 0)),
            pl.BlockSpec((BLK, CPAD), lambda i: (i, 0)),
            pl.BlockSpec((NC, BLK, 16), lambda i: (0, i, 0)),
            pl.BlockSpec((1, CPAD), lambda i: (0, 0)),
        ],
        out_specs=pl.BlockSpec((BLK, CPAD), lambda i: (i, 0)),
        out_shape=jax.ShapeDtypeStruct((NP, CPAD), jnp.float32),
    )(acc2, y2, degp, b2p)


def kernel(x, edge_index, W1, b1, W2, b2):
    src_r = edge_index[0].reshape(NW, 1, EPW)
    dst_r = edge_index[1].reshape(NW, NB, K)
    W2p = jnp.pad(W2, ((0, 0), (0, CPAD - NCLS)))
    b1r = b1.reshape(1, HID)
    b2p = jnp.pad(b2, (0, CPAD - NCLS)).reshape(1, CPAD)

    xp = jnp.pad(x, ((0, NP - N), (0, 0)))
    degp = _sc_degree(dst_r)
    y1 = _tc_layer1(xp, W1, degp)
    acc1 = _sc_aggregate(y1, src_r, dst_r, HID)
    y2 = _tc_layer2_in(acc1, y1, degp, b1r, W2p)
    acc2 = _sc_aggregate(y2, src_r, dst_r, CPAD)
    outp = _tc_final(acc2, y2, degp, b2p)
    return outp[:N, :NCLS]
